# baseline (speedup 1.0000x reference)
import functools

import jax
import jax.numpy as jnp
from jax import lax
from jax.experimental import pallas as pl
from jax.experimental.pallas import tpu as pltpu
from jax.experimental.pallas import tpu_sc as plsc

B = 16384
D = 64
NC = 2
NS = 16
NW = NC * NS
BPW = B // NW
CHUNK = 128
NCHUNK = BPW // CHUNK

_mesh = plsc.VectorSubcoreMesh(core_axis_name="c", subcore_axis_name="s")


@functools.partial(
    pl.kernel,
    mesh=_mesh,
    out_type=jax.ShapeDtypeStruct((NW, NCHUNK, CHUNK, D), jnp.float32),
    scratch_types=[
        pltpu.VMEM((NCHUNK, CHUNK), jnp.int32),
        pltpu.VMEM((NCHUNK, CHUNK, D), jnp.float32),
        pltpu.SemaphoreType.DMA,
    ],
    compiler_params=pltpu.CompilerParams(use_tc_tiling_on_sc=False),
)
def _gather_kernel(idx_hbm, table_hbm, out_hbm, idx_v, rows_v, sem):
    wid = lax.axis_index("s") * NC + lax.axis_index("c")
    pltpu.sync_copy(idx_hbm.at[wid], idx_v)
    gathers = [
        pltpu.async_copy(table_hbm.at[idx_v.at[j]], rows_v.at[j], sem)
        for j in range(NCHUNK)
    ]
    for g in gathers:
        g.wait()
    pltpu.sync_copy(rows_v, out_hbm.at[wid])


def kernel(img_ids, codebook):
    idx = img_ids.astype(jnp.int32).reshape(NW, NCHUNK, CHUNK)
    out = _gather_kernel(idx, codebook)
    return out.reshape(B, D)


# speedup vs baseline: 1.6871x; 1.6871x over previous
import functools

import jax
import jax.numpy as jnp
from jax import lax
from jax.experimental import pallas as pl
from jax.experimental.pallas import tpu as pltpu
from jax.experimental.pallas import tpu_sc as plsc

B = 16384
D = 64
NC = 2
NS = 16
NW = NC * NS
BPW = B // NW
NG = BPW // 16

_mesh = plsc.VectorSubcoreMesh(core_axis_name="c", subcore_axis_name="s")


@functools.partial(
    pl.kernel,
    mesh=_mesh,
    out_type=jax.ShapeDtypeStruct((NW, BPW, D), jnp.float32),
    scratch_types=[
        pltpu.VMEM((BPW,), jnp.int32),
        pltpu.VMEM((BPW, D), jnp.float32),
        pltpu.SemaphoreType.DMA,
    ],
)
def _gather_kernel(idx_hbm, table_hbm, out_hbm, idx_v, rows_v, sem):
    wid = lax.axis_index("s") * NC + lax.axis_index("c")
    pltpu.sync_copy(idx_hbm.at[wid], idx_v)

    def fire_group(g):
        vec = idx_v[pl.ds(g * 16, 16)]
        base = g * 16
        for l in range(16):
            pltpu.make_async_copy(
                table_hbm.at[vec[l]], rows_v.at[base + l], sem
            ).start()

    def drain_group():
        for _ in range(16):
            pltpu.make_async_copy(table_hbm.at[0], rows_v.at[0], sem).wait()

    def body(g, carry):
        fire_group(g)
        drain_group()
        return carry

    fire_group(0)
    lax.fori_loop(1, NG, body, 0)
    drain_group()

    pltpu.sync_copy(rows_v, out_hbm.at[wid])


def kernel(img_ids, codebook):
    idx = img_ids.astype(jnp.int32).reshape(NW, BPW)
    out = _gather_kernel(idx, codebook)
    return out.reshape(B, D)
